# baseline (speedup 1.0000x reference)
import functools

import jax
import jax.numpy as jnp
from jax import lax
from jax.experimental import pallas as pl
from jax.experimental.pallas import tpu as pltpu
from jax.experimental.pallas import tpu_sc as plsc

_N, _E, _D = 10000, 320000, 128
_NW = 32
_C = 128
_K = 81
_K0 = 150
_K1 = 12
_EPAD = _NW * _K * _C
_NACC = 10112
_RPS = _NACC // 16
_BLK = 1264
_GRID = _NACC // _BLK

_mesh = plsc.VectorSubcoreMesh(core_axis_name="c", subcore_axis_name="s")


def _sc_degree(dst_r, ones_d, zeros_d):
    @functools.partial(
        pl.kernel,
        mesh=_mesh,
        out_type=jax.ShapeDtypeStruct((2 * _NACC, _D), jnp.float32),
        scratch_types=[
            pltpu.VMEM((_K, _C), jnp.int32),
            pltpu.VMEM((_C, _D), jnp.float32),
            pltpu.VMEM_SHARED((_NACC, _D), jnp.float32),
            pltpu.SemaphoreType.DMA,
        ],
    )
    def run(dst_hbm, ones_hbm, zeros_hbm, out_hbm, dst_v, ones_v, dacc, sem):
        c = lax.axis_index("c")
        s = lax.axis_index("s")
        wid = c * 16 + s
        pltpu.async_copy(dst_hbm.at[wid], dst_v, sem).wait()
        pltpu.async_copy(ones_hbm, ones_v, sem).wait()
        pltpu.async_copy(zeros_hbm, dacc.at[pl.ds(s * _RPS, _RPS)], sem).wait()
        plsc.subcore_barrier()

        @pl.loop(0, _K)
        def _(j):
            pltpu.sync_copy(ones_v, dacc.at[dst_v.at[j]], add=True)

        plsc.subcore_barrier()
        pltpu.async_copy(
            dacc.at[pl.ds(s * _RPS, _RPS)],
            out_hbm.at[pl.ds(c * _NACC + s * _RPS, _RPS)],
            sem,
        ).wait()

    return run(dst_r, ones_d, zeros_d)


def _sc_scatter(hp, e_r, zeros_d):
    @functools.partial(
        pl.kernel,
        mesh=_mesh,
        out_type=jax.ShapeDtypeStruct((2 * _NACC, _D), jnp.float32),
        scratch_types=[
            pltpu.VMEM((2, _C), jnp.int32),
            pltpu.VMEM((2, _C), jnp.int32),
            pltpu.VMEM((2, _C), jnp.int32),
            pltpu.VMEM((_C, _D), jnp.float32),
            pltpu.VMEM((_C, _D), jnp.float32),
            pltpu.VMEM((_C, _D), jnp.float32),
            pltpu.VMEM_SHARED((_NACC, _D), jnp.float32),
            pltpu.SemaphoreType.DMA,
            pltpu.SemaphoreType.DMA,
            pltpu.SemaphoreType.DMA,
            pltpu.SemaphoreType.DMA,
            pltpu.SemaphoreType.DMA,
            pltpu.SemaphoreType.DMA,
            pltpu.SemaphoreType.DMA,
        ],
    )
    def run(hp_hbm, e_hbm, zeros_hbm, out_hbm,
            ix0, ix1, ix2, rows0, rows1, rows2, acc,
            g0, g1, g2, i0, i1, i2, zsem):
        c = lax.axis_index("c")
        s = lax.axis_index("s")
        base = jnp.where(c == 0, s * _K0, 16 * _K0 + s * _K1)
        k_third = jnp.where(c == 0, _K0 // 3, _K1 // 3)
        pltpu.async_copy(zeros_hbm, acc.at[pl.ds(s * _RPS, _RPS)], zsem)
        ia = pltpu.async_copy(e_hbm.at[base], ix0, i0)
        ib = pltpu.async_copy(e_hbm.at[base + 1], ix1, i1)
        pltpu.async_copy(e_hbm.at[base + 2], ix2, i2)
        pltpu.make_async_copy(zeros_hbm, acc.at[pl.ds(s * _RPS, _RPS)], zsem).wait()
        plsc.subcore_barrier()

        ia.wait()
        pltpu.async_copy(hp_hbm.at[ix0.at[0]], rows0, g0)
        ib.wait()
        pltpu.async_copy(hp_hbm.at[ix1.at[0]], rows1, g1)

        @pl.loop(0, k_third)
        def _(t):
            j0 = base + 3 * t
            pltpu.make_async_copy(hp_hbm.at[ix0.at[0]], rows0, g0).wait()
            pltpu.make_async_copy(e_hbm.at[j0 + 2], ix2, i2).wait()
            h2 = pltpu.async_copy(hp_hbm.at[ix2.at[0]], rows2, g2)
            pltpu.sync_copy(rows0, acc.at[ix0.at[1]], add=True)

            @pl.when(t < k_third - 1)
            def _():
                nia = pltpu.async_copy(e_hbm.at[j0 + 3], ix0, i0)
                nia.wait()
                pltpu.async_copy(hp_hbm.at[ix0.at[0]], rows0, g0)

            pltpu.make_async_copy(hp_hbm.at[ix1.at[0]], rows1, g1).wait()
            pltpu.sync_copy(rows1, acc.at[ix1.at[1]], add=True)

            @pl.when(t < k_third - 1)
            def _():
                nib = pltpu.async_copy(e_hbm.at[j0 + 4], ix1, i1)
                nib.wait()
                pltpu.async_copy(hp_hbm.at[ix1.at[0]], rows1, g1)

            h2.wait()
            pltpu.sync_copy(rows2, acc.at[ix2.at[1]], add=True)

            @pl.when(t < k_third - 1)
            def _():
                pltpu.async_copy(e_hbm.at[j0 + 5], ix2, i2)

        plsc.subcore_barrier()
        pltpu.async_copy(
            acc.at[pl.ds(s * _RPS, _RPS)],
            out_hbm.at[pl.ds(c * _NACC + s * _RPS, _RPS)],
            g0,
        ).wait()

    return run(hp, e_r, zeros_d)


def _dot(a, b):
    return jax.lax.dot_general(
        a, b, (((1,), (0,)), ((), ())),
        preferred_element_type=jnp.float32,
        precision=None,
    )


def _tc_encode(xp, W_enc, b_enc):
    def body(x_ref, w_ref, b_ref, o_ref):
        o_ref[...] = _dot(x_ref[...], w_ref[...]) + b_ref[...]

    return pl.pallas_call(
        body,
        grid=(_GRID,),
        in_specs=[
            pl.BlockSpec((_BLK, _D), lambda i: (i, 0)),
            pl.BlockSpec((_D, _D), lambda i: (0, 0)),
            pl.BlockSpec((1, _D), lambda i: (0, 0)),
        ],
        out_specs=pl.BlockSpec((_BLK, _D), lambda i: (i, 0)),
        out_shape=jax.ShapeDtypeStruct((_NACC, _D), jnp.float32),
    )(xp, W_enc, b_enc)


def _tc_prescale(henc, degp, W0):
    def body(h_ref, d_ref, w_ref, hp_ref, dinv_ref):
        deg = d_ref[0, :, 0:1] + d_ref[1, :, 0:1] + 1.0
        d = jnp.maximum(deg, 1.0)
        r = lax.rsqrt(d)
        dinv = r * (1.5 - 0.5 * d * r * r)
        dinv_ref[...] = dinv
        hp_ref[...] = dinv * _dot(h_ref[...], w_ref[...])

    return pl.pallas_call(
        body,
        grid=(_GRID,),
        in_specs=[
            pl.BlockSpec((_BLK, _D), lambda i: (i, 0)),
            pl.BlockSpec((2, _BLK, _D), lambda i: (0, i, 0)),
            pl.BlockSpec((_D, _D), lambda i: (0, 0)),
        ],
        out_specs=[
            pl.BlockSpec((_BLK, _D), lambda i: (i, 0)),
            pl.BlockSpec((_BLK, 1), lambda i: (i, 0)),
        ],
        out_shape=[
            jax.ShapeDtypeStruct((_NACC, _D), jnp.float32),
            jax.ShapeDtypeStruct((_NACC, 1), jnp.float32),
        ],
    )(henc, degp, W0)


def _tc_layer(parts, hp, dinv, b, W_next):
    def body(p_ref, hp_ref, dinv_ref, b_ref, w_ref, o_ref):
        dinv = dinv_ref[...]
        z = dinv * (p_ref[0] + p_ref[1] + hp_ref[...]) + b_ref[...]
        a = jnp.maximum(z, 0.0)
        o_ref[...] = dinv * _dot(a, w_ref[...])

    return pl.pallas_call(
        body,
        grid=(_GRID,),
        in_specs=[
            pl.BlockSpec((2, _BLK, _D), lambda i: (0, i, 0)),
            pl.BlockSpec((_BLK, _D), lambda i: (i, 0)),
            pl.BlockSpec((_BLK, 1), lambda i: (i, 0)),
            pl.BlockSpec((1, _D), lambda i: (0, 0)),
            pl.BlockSpec((_D, _D), lambda i: (0, 0)),
        ],
        out_specs=pl.BlockSpec((_BLK, _D), lambda i: (i, 0)),
        out_shape=jax.ShapeDtypeStruct((_NACC, _D), jnp.float32),
    )(parts, hp, dinv, b, W_next)


def _tc_readout(parts, hp, dinv, b2, Wr0, br0, Wr1, br1, Wr2, br2):
    def body(p_ref, hp_ref, dinv_ref, b2_ref,
             w0_ref, c0_ref, w1_ref, c1_ref, w2_ref, c2_ref, o_ref):
        z = dinv_ref[...] * (p_ref[0] + p_ref[1] + hp_ref[...]) + b2_ref[...]
        y = jnp.maximum(_dot(z, w0_ref[...]) + c0_ref[...], 0.0)
        y = jnp.maximum(_dot(y, w1_ref[...]) + c1_ref[...], 0.0)
        o_ref[...] = _dot(y, w2_ref[...]) + c2_ref[...]

    return pl.pallas_call(
        body,
        grid=(_GRID,),
        in_specs=[
            pl.BlockSpec((2, _BLK, _D), lambda i: (0, i, 0)),
            pl.BlockSpec((_BLK, _D), lambda i: (i, 0)),
            pl.BlockSpec((_BLK, 1), lambda i: (i, 0)),
            pl.BlockSpec((1, _D), lambda i: (0, 0)),
            pl.BlockSpec((_D, 64), lambda i: (0, 0)),
            pl.BlockSpec((1, 64), lambda i: (0, 0)),
            pl.BlockSpec((64, 32), lambda i: (0, 0)),
            pl.BlockSpec((1, 32), lambda i: (0, 0)),
            pl.BlockSpec((32, 1), lambda i: (0, 0)),
            pl.BlockSpec((1, 1), lambda i: (0, 0)),
        ],
        out_specs=pl.BlockSpec((_BLK, 1), lambda i: (i, 0)),
        out_shape=jax.ShapeDtypeStruct((_NACC, 1), jnp.float32),
    )(parts, hp, dinv, b2, Wr0, br0, Wr1, br1, Wr2, br2)


def kernel(x, edge_index, W_enc, b_enc, W0, b0, W1, b1, W2, b2,
           Wr0, br0, Wr1, br1, Wr2, br2):
    pad_e = _EPAD - _E
    src_r = jnp.pad(edge_index[0], (0, pad_e), constant_values=_N)
    src_r = src_r.reshape(_NW, _K, _C)
    dst_r = jnp.pad(edge_index[1], (0, pad_e), constant_values=_N)
    dst_r = dst_r.reshape(_NW, _K, _C)
    e_r = jnp.stack((src_r, dst_r), axis=2).reshape(_NW * _K, 2, _C)
    xp = jnp.pad(x, ((0, _NACC - _N), (0, 0)))

    ones_d = jnp.ones((_C, _D), jnp.float32)
    zeros_d = jnp.zeros((_RPS, _D), jnp.float32)

    b_enc2 = b_enc.reshape(1, _D)
    b0_2, b1_2, b2_2 = b0.reshape(1, _D), b1.reshape(1, _D), b2.reshape(1, _D)
    br0_2, br1_2, br2_2 = br0.reshape(1, 64), br1.reshape(1, 32), br2.reshape(1, 1)

    degp = _sc_degree(dst_r, ones_d, zeros_d).reshape(2, _NACC, _D)
    henc = _tc_encode(xp, W_enc, b_enc2)

    hp0, dinv = _tc_prescale(henc, degp, W0)

    p0 = _sc_scatter(hp0, e_r, zeros_d).reshape(2, _NACC, _D)
    hp1 = _tc_layer(p0, hp0, dinv, b0_2, W1)

    p1 = _sc_scatter(hp1, e_r, zeros_d).reshape(2, _NACC, _D)
    hp2 = _tc_layer(p1, hp1, dinv, b1_2, W2)

    p2 = _sc_scatter(hp2, e_r, zeros_d).reshape(2, _NACC, _D)
    y = _tc_readout(p2, hp2, dinv, b2_2, Wr0, br0_2, Wr1, br1_2, Wr2, br2_2)

    return y[:_N]


# speedup vs baseline: 1.3520x; 1.3520x over previous
import functools

import jax
import jax.numpy as jnp
from jax import lax
from jax.experimental import pallas as pl
from jax.experimental.pallas import tpu as pltpu
from jax.experimental.pallas import tpu_sc as plsc

_N, _E, _D = 10000, 320000, 128
_NW = 32
_C = 128
_K = 80
_K0 = 148
_K1 = 12
_EPAD = _NW * _K * _C
_NACC = 10240
_RPS = _NACC // 16
_BLK = 1280
_GRID = _NACC // _BLK

_mesh = plsc.VectorSubcoreMesh(core_axis_name="c", subcore_axis_name="s")


def _sc_degree(dst_r, ones_d, zeros_d):
    @functools.partial(
        pl.kernel,
        mesh=_mesh,
        out_type=jax.ShapeDtypeStruct((2 * _NACC, _D), jnp.float32),
        scratch_types=[
            pltpu.VMEM((_K, _C), jnp.int32),
            pltpu.VMEM((_C, _D), jnp.float32),
            pltpu.VMEM_SHARED((_NACC, _D), jnp.float32),
            pltpu.SemaphoreType.DMA,
        ],
    )
    def run(dst_hbm, ones_hbm, zeros_hbm, out_hbm, dst_v, ones_v, dacc, sem):
        c = lax.axis_index("c")
        s = lax.axis_index("s")
        wid = c * 16 + s
        pltpu.async_copy(dst_hbm.at[wid], dst_v, sem).wait()
        pltpu.async_copy(ones_hbm, ones_v, sem).wait()
        pltpu.async_copy(zeros_hbm, dacc.at[pl.ds(s * _RPS, _RPS)], sem).wait()
        plsc.subcore_barrier()

        @pl.loop(0, _K)
        def _(j):
            pltpu.sync_copy(ones_v, dacc.at[dst_v.at[j]], add=True)

        plsc.subcore_barrier()
        pltpu.async_copy(
            dacc.at[pl.ds(s * _RPS, _RPS)],
            out_hbm.at[pl.ds(c * _NACC + s * _RPS, _RPS)],
            sem,
        ).wait()

    return run(dst_r, ones_d, zeros_d)


def _sc_scatter(hp, e_r, zeros_d):
    @functools.partial(
        pl.kernel,
        mesh=_mesh,
        out_type=jax.ShapeDtypeStruct((2 * _NACC, _D), jnp.float32),
        scratch_types=[
            pltpu.VMEM((2, _C), jnp.int32),
            pltpu.VMEM((2, _C), jnp.int32),
            pltpu.VMEM((_C, _D), jnp.float32),
            pltpu.VMEM((_C, _D), jnp.float32),
            pltpu.VMEM_SHARED((_NACC, _D), jnp.float32),
            pltpu.SemaphoreType.DMA,
            pltpu.SemaphoreType.DMA,
            pltpu.SemaphoreType.DMA,
            pltpu.SemaphoreType.DMA,
            pltpu.SemaphoreType.DMA,
        ],
    )
    def run(hp_hbm, e_hbm, zeros_hbm, out_hbm,
            ixa, ixb, rows0, rows1, acc, gsem0, gsem1, isema, isemb, zsem):
        c = lax.axis_index("c")
        s = lax.axis_index("s")
        base = jnp.where(c == 0, s * _K0, 16 * _K0 + s * _K1)
        k_half = jnp.where(c == 0, _K0 // 2, _K1 // 2)
        pltpu.async_copy(zeros_hbm, acc.at[pl.ds(s * _RPS, _RPS)], zsem)
        pltpu.async_copy(e_hbm.at[base], ixa, isema).wait()
        pltpu.async_copy(e_hbm.at[base + 1], ixb, isemb)
        pltpu.make_async_copy(zeros_hbm, acc.at[pl.ds(s * _RPS, _RPS)], zsem).wait()
        plsc.subcore_barrier()

        pltpu.async_copy(hp_hbm.at[ixa.at[0]], rows0, gsem0)

        @pl.loop(0, k_half)
        def _(t):
            j0 = base + 2 * t
            pltpu.make_async_copy(hp_hbm.at[ixa.at[0]], rows0, gsem0).wait()
            pltpu.make_async_copy(e_hbm.at[j0 + 1], ixb, isemb).wait()
            pltpu.async_copy(hp_hbm.at[ixb.at[0]], rows1, gsem1)
            pltpu.sync_copy(rows0, acc.at[ixa.at[1]], add=True)

            @pl.when(t < k_half - 1)
            def _():
                ia = pltpu.async_copy(e_hbm.at[j0 + 2], ixa, isema)
                pltpu.make_async_copy(hp_hbm.at[ixb.at[0]], rows1, gsem1).wait()
                ia.wait()
                pltpu.async_copy(hp_hbm.at[ixa.at[0]], rows0, gsem0)
                pltpu.sync_copy(rows1, acc.at[ixb.at[1]], add=True)
                pltpu.async_copy(e_hbm.at[j0 + 3], ixb, isemb)

            @pl.when(t == k_half - 1)
            def _():
                pltpu.make_async_copy(hp_hbm.at[ixb.at[0]], rows1, gsem1).wait()
                pltpu.sync_copy(rows1, acc.at[ixb.at[1]], add=True)

        plsc.subcore_barrier()
        pltpu.async_copy(
            acc.at[pl.ds(s * _RPS, _RPS)],
            out_hbm.at[pl.ds(c * _NACC + s * _RPS, _RPS)],
            gsem0,
        ).wait()

    return run(hp, e_r, zeros_d)


def _dot(a, b):
    return jax.lax.dot_general(
        a, b, (((1,), (0,)), ((), ())),
        preferred_element_type=jnp.float32,
        precision=None,
    )


def _tc_encode(xp, W_enc, b_enc):
    def body(x_ref, w_ref, b_ref, o_ref):
        o_ref[...] = _dot(x_ref[...], w_ref[...]) + b_ref[...]

    return pl.pallas_call(
        body,
        grid=(_GRID,),
        in_specs=[
            pl.BlockSpec((_BLK, _D), lambda i: (i, 0)),
            pl.BlockSpec((_D, _D), lambda i: (0, 0)),
            pl.BlockSpec((1, _D), lambda i: (0, 0)),
        ],
        out_specs=pl.BlockSpec((_BLK, _D), lambda i: (i, 0)),
        out_shape=jax.ShapeDtypeStruct((_NACC, _D), jnp.float32),
    )(xp, W_enc, b_enc)


def _tc_prescale(henc, degp, W0):
    def body(h_ref, d_ref, w_ref, hp_ref, dinv_ref):
        deg = d_ref[0, :, 0:1] + d_ref[1, :, 0:1] + 1.0
        dinv = lax.rsqrt(jnp.maximum(deg, 1.0))
        dinv_ref[...] = dinv
        hp_ref[...] = dinv * _dot(h_ref[...], w_ref[...])

    return pl.pallas_call(
        body,
        grid=(_GRID,),
        in_specs=[
            pl.BlockSpec((_BLK, _D), lambda i: (i, 0)),
            pl.BlockSpec((2, _BLK, _D), lambda i: (0, i, 0)),
            pl.BlockSpec((_D, _D), lambda i: (0, 0)),
        ],
        out_specs=[
            pl.BlockSpec((_BLK, _D), lambda i: (i, 0)),
            pl.BlockSpec((_BLK, 1), lambda i: (i, 0)),
        ],
        out_shape=[
            jax.ShapeDtypeStruct((_NACC, _D), jnp.float32),
            jax.ShapeDtypeStruct((_NACC, 1), jnp.float32),
        ],
    )(henc, degp, W0)


def _tc_layer(parts, hp, dinv, b, W_next):
    def body(p_ref, hp_ref, dinv_ref, b_ref, w_ref, o_ref):
        dinv = dinv_ref[...]
        z = dinv * (p_ref[0] + p_ref[1] + hp_ref[...]) + b_ref[...]
        a = jnp.maximum(z, 0.0)
        o_ref[...] = dinv * _dot(a, w_ref[...])

    return pl.pallas_call(
        body,
        grid=(_GRID,),
        in_specs=[
            pl.BlockSpec((2, _BLK, _D), lambda i: (0, i, 0)),
            pl.BlockSpec((_BLK, _D), lambda i: (i, 0)),
            pl.BlockSpec((_BLK, 1), lambda i: (i, 0)),
            pl.BlockSpec((1, _D), lambda i: (0, 0)),
            pl.BlockSpec((_D, _D), lambda i: (0, 0)),
        ],
        out_specs=pl.BlockSpec((_BLK, _D), lambda i: (i, 0)),
        out_shape=jax.ShapeDtypeStruct((_NACC, _D), jnp.float32),
    )(parts, hp, dinv, b, W_next)


def _tc_readout(parts, hp, dinv, b2, Wr0, br0, Wr1, br1, Wr2, br2):
    def body(p_ref, hp_ref, dinv_ref, b2_ref,
             w0_ref, c0_ref, w1_ref, c1_ref, w2_ref, c2_ref, o_ref):
        z = dinv_ref[...] * (p_ref[0] + p_ref[1] + hp_ref[...]) + b2_ref[...]
        y = jnp.maximum(_dot(z, w0_ref[...]) + c0_ref[...], 0.0)
        y = jnp.maximum(_dot(y, w1_ref[...]) + c1_ref[...], 0.0)
        o_ref[...] = _dot(y, w2_ref[...]) + c2_ref[...]

    return pl.pallas_call(
        body,
        grid=(_GRID,),
        in_specs=[
            pl.BlockSpec((2, _BLK, _D), lambda i: (0, i, 0)),
            pl.BlockSpec((_BLK, _D), lambda i: (i, 0)),
            pl.BlockSpec((_BLK, 1), lambda i: (i, 0)),
            pl.BlockSpec((1, _D), lambda i: (0, 0)),
            pl.BlockSpec((_D, 64), lambda i: (0, 0)),
            pl.BlockSpec((1, 64), lambda i: (0, 0)),
            pl.BlockSpec((64, 32), lambda i: (0, 0)),
            pl.BlockSpec((1, 32), lambda i: (0, 0)),
            pl.BlockSpec((32, 1), lambda i: (0, 0)),
            pl.BlockSpec((1, 1), lambda i: (0, 0)),
        ],
        out_specs=pl.BlockSpec((_BLK, 1), lambda i: (i, 0)),
        out_shape=jax.ShapeDtypeStruct((_NACC, 1), jnp.float32),
    )(parts, hp, dinv, b2, Wr0, br0, Wr1, br1, Wr2, br2)


def kernel(x, edge_index, W_enc, b_enc, W0, b0, W1, b1, W2, b2,
           Wr0, br0, Wr1, br1, Wr2, br2):
    pad_e = _EPAD - _E
    src_r = jnp.pad(edge_index[0], (0, pad_e), constant_values=_N)
    src_r = src_r.reshape(_NW, _K, _C)
    dst_r = jnp.pad(edge_index[1], (0, pad_e), constant_values=_N)
    dst_r = dst_r.reshape(_NW, _K, _C)
    e_r = jnp.stack((src_r, dst_r), axis=2).reshape(_NW * _K, 2, _C)
    xp = jnp.pad(x, ((0, _NACC - _N), (0, 0)))

    ones_d = jnp.ones((_C, _D), jnp.float32)
    zeros_d = jnp.zeros((_RPS, _D), jnp.float32)

    b_enc2 = b_enc.reshape(1, _D)
    b0_2, b1_2, b2_2 = b0.reshape(1, _D), b1.reshape(1, _D), b2.reshape(1, _D)
    br0_2, br1_2, br2_2 = br0.reshape(1, 64), br1.reshape(1, 32), br2.reshape(1, 1)

    degp = _sc_degree(dst_r, ones_d, zeros_d).reshape(2, _NACC, _D)
    henc = _tc_encode(xp, W_enc, b_enc2)

    hp0, dinv = _tc_prescale(henc, degp, W0)

    p0 = _sc_scatter(hp0, e_r, zeros_d).reshape(2, _NACC, _D)
    hp1 = _tc_layer(p0, hp0, dinv, b0_2, W1)

    p1 = _sc_scatter(hp1, e_r, zeros_d).reshape(2, _NACC, _D)
    hp2 = _tc_layer(p1, hp1, dinv, b1_2, W2)

    p2 = _sc_scatter(hp2, e_r, zeros_d).reshape(2, _NACC, _D)
    y = _tc_readout(p2, hp2, dinv, b2_2, Wr0, br0_2, Wr1, br1_2, Wr2, br2_2)

    return y[:_N]
